# baseline (speedup 1.0000x reference)
import functools

import jax
import jax.numpy as jnp
from jax import lax
from jax.experimental import pallas as pl
from jax.experimental.pallas import tpu as pltpu
from jax.experimental.pallas import tpu_sc as plsc

_NC = 2
_NS = 16
_NW = _NC * _NS
_L = 16


def _build(b, s, d, dtype):
    bw = b // _NW
    nr = d // 8

    mesh = plsc.VectorSubcoreMesh(core_axis_name="c", subcore_axis_name="s")

    @functools.partial(
        pl.kernel,
        out_type=jax.ShapeDtypeStruct((s, nr, _NW, 8, bw), dtype),
        mesh=mesh,
        scratch_types=[
            pltpu.VMEM((bw, s), jnp.int32),
            pltpu.VMEM((s, bw), jnp.int32),
            pltpu.VMEM((bw, d), dtype),
            pltpu.VMEM((bw, d), dtype),
            pltpu.VMEM((nr, 8, bw), dtype),
            pltpu.VMEM((nr, 8, bw), dtype),
            [pltpu.SemaphoreType.DMA] * 2,
            [pltpu.SemaphoreType.DMA] * 2,
        ],
        compiler_params=pltpu.CompilerParams(
            use_tc_tiling_on_sc=False, needs_layout_passes=False
        ),
    )
    def emb(idx_hbm, table_hbm, out_hbm, xb, xbt, r0, r1, t0, t1, gsems, wsems):
        wid = lax.axis_index("s") * _NC + lax.axis_index("c")
        rows = (r0, r1)
        tiles = (t0, t1)
        iota = lax.iota(jnp.int32, _L)

        pltpu.sync_copy(idx_hbm.at[pl.ds(wid * bw, bw)], xb)

        @pl.loop(0, s)
        def _transpose(j):
            jv = jnp.full((_L,), j, jnp.int32)
            for m in range(bw // _L):
                v = plsc.load_gather(xb, [iota + m * _L, jv])
                xbt[j, pl.ds(m * _L, _L)] = v

        pltpu.async_copy(table_hbm.at[xbt.at[0]], rows[0], gsems[0])

        @pl.loop(0, s, step=2)
        def _unit(g):
            for u in range(2):
                j = g + u
                pltpu.make_async_copy(
                    table_hbm.at[xbt.at[j]], rows[u], gsems[u]
                ).wait()

                @pl.when(j + 1 < s)
                def _prefetch():
                    pltpu.async_copy(
                        table_hbm.at[xbt.at[j + 1]], rows[1 - u], gsems[1 - u]
                    )

                @pl.when(j >= 2)
                def _drain():
                    pltpu.make_async_copy(
                        tiles[u], out_hbm.at[j - 2, :, wid], wsems[u]
                    ).wait()

                @pl.loop(0, nr)
                def _assemble(q):
                    for r in range(8):
                        dv = jnp.full((_L,), 0, jnp.int32) + (q * 8 + r)
                        for k in range(bw // _L):
                            v = plsc.load_gather(
                                rows[u], [iota + k * _L, dv]
                            )
                            tiles[u][q, r, pl.ds(k * _L, _L)] = v

                pltpu.async_copy(tiles[u], out_hbm.at[j, :, wid], wsems[u])

        for u in range(2):
            pltpu.make_async_copy(
                tiles[u], out_hbm.at[s - 2 + u, :, wid], wsems[u]
            ).wait()

    return emb


def kernel(x, table):
    b, s = x.shape
    v, d = table.shape
    out_pal = _build(b, s, d, table.dtype)(x.astype(jnp.int32), table)
    return out_pal.transpose(2, 4, 0, 1, 3).reshape(b, s, d)


# speedup vs baseline: 1.1478x; 1.1478x over previous
import functools

import jax
import jax.numpy as jnp
from jax import lax
from jax.experimental import pallas as pl
from jax.experimental.pallas import tpu as pltpu
from jax.experimental.pallas import tpu_sc as plsc

_NC = 2
_NS = 16
_NW = _NC * _NS
_L = 16


def _build(b, s, d, dtype):
    bw = b // _NW
    nr = d // 8

    mesh = plsc.VectorSubcoreMesh(core_axis_name="c", subcore_axis_name="s")

    @functools.partial(
        pl.kernel,
        out_type=jax.ShapeDtypeStruct((s, nr, _NW, 8, bw), dtype),
        mesh=mesh,
        scratch_types=[
            pltpu.VMEM((s, bw), jnp.int32),
            pltpu.VMEM((bw, d), dtype),
            pltpu.VMEM((bw, d), dtype),
            pltpu.VMEM((nr, 8, bw), dtype),
            pltpu.VMEM((nr, 8, bw), dtype),
            [pltpu.SemaphoreType.DMA] * 2,
            [pltpu.SemaphoreType.DMA] * 2,
        ],
        compiler_params=pltpu.CompilerParams(
            use_tc_tiling_on_sc=False, needs_layout_passes=False
        ),
    )
    def emb(idx_hbm, table_hbm, out_hbm, xbt, r0, r1, t0, t1, gsems, wsems):
        wid = lax.axis_index("s") * _NC + lax.axis_index("c")
        rows = (r0, r1)
        tiles = (t0, t1)
        iota = lax.iota(jnp.int32, _L)
        qv = [(iota + m * _L) // 8 for m in range(d // _L)]
        rv = [(iota + m * _L) % 8 for m in range(d // _L)]

        pltpu.sync_copy(idx_hbm.at[:, pl.ds(wid * bw, bw)], xbt)
        pltpu.async_copy(table_hbm.at[xbt.at[0]], rows[0], gsems[0])

        @pl.loop(0, s, step=2)
        def _unit(g):
            for u in range(2):
                j = g + u
                pltpu.make_async_copy(
                    table_hbm.at[xbt.at[j]], rows[u], gsems[u]
                ).wait()

                @pl.when(j + 1 < s)
                def _prefetch():
                    pltpu.async_copy(
                        table_hbm.at[xbt.at[j + 1]], rows[1 - u], gsems[1 - u]
                    )

                @pl.when(j >= 2)
                def _drain():
                    pltpu.make_async_copy(
                        tiles[u], out_hbm.at[j - 2, :, wid], wsems[u]
                    ).wait()

                for c in range(bw):
                    cv = jnp.full((_L,), c, jnp.int32)
                    for m in range(d // _L):
                        v = rows[u][c, pl.ds(m * _L, _L)]
                        plsc.store_scatter(tiles[u], [qv[m], rv[m], cv], v)

                pltpu.async_copy(tiles[u], out_hbm.at[j, :, wid], wsems[u])

        for u in range(2):
            pltpu.make_async_copy(
                tiles[u], out_hbm.at[s - 2 + u, :, wid], wsems[u]
            ).wait()

    return emb


def kernel(x, table):
    b, s = x.shape
    v, d = table.shape
    xt = jnp.swapaxes(x, 0, 1).astype(jnp.int32)
    out_pal = _build(b, s, d, table.dtype)(xt, table)
    return out_pal.transpose(2, 4, 0, 1, 3).reshape(b, s, d)


# speedup vs baseline: 1.1570x; 1.0081x over previous
import functools

import jax
import jax.numpy as jnp
from jax import lax
from jax.experimental import pallas as pl
from jax.experimental.pallas import tpu as pltpu
from jax.experimental.pallas import tpu_sc as plsc

_NC = 2
_NS = 16
_NW = _NC * _NS
_L = 16


def _build(b, s, d, dtype):
    bw = b // _NW
    nr = d // 8

    mesh = plsc.VectorSubcoreMesh(core_axis_name="c", subcore_axis_name="s")

    @functools.partial(
        pl.kernel,
        out_type=jax.ShapeDtypeStruct((s, nr, _NW, 8, bw), dtype),
        mesh=mesh,
        scratch_types=[
            pltpu.VMEM((s, bw), jnp.int32),
            [pltpu.VMEM((bw, d), dtype)] * 4,
            [pltpu.VMEM((nr, 8, bw), dtype)] * 2,
            [pltpu.SemaphoreType.DMA] * 4,
            [pltpu.SemaphoreType.DMA] * 2,
        ],
        compiler_params=pltpu.CompilerParams(
            use_tc_tiling_on_sc=False, needs_layout_passes=False
        ),
    )
    def emb(idx_hbm, table_hbm, out_hbm, xbt, rows, tiles, gsems, wsems):
        wid = lax.axis_index("s") * _NC + lax.axis_index("c")
        iota = lax.iota(jnp.int32, _L)
        qv = [(iota + m * _L) // 8 for m in range(d // _L)]
        rv = [(iota + m * _L) % 8 for m in range(d // _L)]
        nbuf = 4

        def assemble(src, dst):
            @pl.loop(0, bw, step=8)
            def _c(c0):
                for cc in range(8):
                    c = c0 + cc
                    cv = jnp.full((_L,), 0, jnp.int32) + c
                    for m in range(d // _L):
                        v = src[c, pl.ds(m * _L, _L)]
                        plsc.store_scatter(dst, [qv[m], rv[m], cv], v)

        pltpu.sync_copy(idx_hbm.at[:, pl.ds(wid * bw, bw)], xbt)
        for u in range(nbuf):
            pltpu.async_copy(table_hbm.at[xbt.at[u]], rows[u], gsems[u])

        @pl.loop(0, s, step=nbuf)
        def _group(g):
            for u in range(nbuf):
                j = g + u
                t = u % 2
                pltpu.make_async_copy(
                    table_hbm.at[xbt.at[j]], rows[u], gsems[u]
                ).wait()

                @pl.when(j >= 2)
                def _drain():
                    pltpu.make_async_copy(
                        tiles[t], out_hbm.at[j - 2, :, wid], wsems[t]
                    ).wait()

                assemble(rows[u], tiles[t])
                pltpu.async_copy(tiles[t], out_hbm.at[j, :, wid], wsems[t])

                @pl.when(j + nbuf < s)
                def _prefetch():
                    pltpu.async_copy(
                        table_hbm.at[xbt.at[j + nbuf]], rows[u], gsems[u]
                    )

        for u in range(2):
            pltpu.make_async_copy(
                tiles[u], out_hbm.at[s - 2 + u, :, wid], wsems[u]
            ).wait()

    return emb


def kernel(x, table):
    b, s = x.shape
    v, d = table.shape
    xt = jnp.swapaxes(x, 0, 1).astype(jnp.int32)
    out_pal = _build(b, s, d, table.dtype)(xt, table)
    return out_pal.transpose(2, 4, 0, 1, 3).reshape(b, s, d)


# speedup vs baseline: 1.4730x; 1.2731x over previous
import functools

import jax
import jax.numpy as jnp
from jax import lax
from jax.experimental import pallas as pl
from jax.experimental.pallas import tpu as pltpu
from jax.experimental.pallas import tpu_sc as plsc

_NC = 2
_NS = 16
_NW = _NC * _NS


def _build(b, s, d, v, dtype):
    b_per_w = b // _NW
    nbuf = 2

    mesh = plsc.VectorSubcoreMesh(core_axis_name="c", subcore_axis_name="s")

    @functools.partial(
        pl.kernel,
        out_type=jax.ShapeDtypeStruct((b, s, d), dtype),
        mesh=mesh,
        scratch_types=[
            pltpu.VMEM((b_per_w, s), jnp.int32),
            [pltpu.VMEM((s, 2 * d), dtype)] * 2,
            [pltpu.SemaphoreType.DMA] * nbuf,
        ],
        compiler_params=pltpu.CompilerParams(
            use_tc_tiling_on_sc=False, needs_layout_passes=False
        ),
    )
    def emb(idx_hbm, table2_hbm, out_hbm, idx_v, rows, gsems):
        wid = lax.axis_index("s") * _NC + lax.axis_index("c")
        base = wid * b_per_w
        pltpu.sync_copy(idx_hbm.at[pl.ds(base, b_per_w)], idx_v)

        for u in range(nbuf):
            pltpu.async_copy(table2_hbm.at[idx_v.at[u]], rows[u], gsems[u])

        @pl.loop(0, b_per_w, step=nbuf)
        def _group(g):
            for u in range(nbuf):
                j = g + u
                pltpu.make_async_copy(
                    table2_hbm.at[idx_v.at[j]], rows[u], gsems[u]
                ).wait()
                pltpu.sync_copy(rows[u].at[:, pl.ds(0, d)], out_hbm.at[base + j])

                @pl.when(j + nbuf < b_per_w)
                def _prefetch():
                    pltpu.async_copy(
                        table2_hbm.at[idx_v.at[j + nbuf]], rows[u], gsems[u]
                    )

    return emb


def kernel(x, table):
    b, s = x.shape
    v, d = table.shape
    table2 = jnp.pad(table, ((0, 0), (0, d)))
    return _build(b, s, d, v, table.dtype)(x.astype(jnp.int32), table2)


# speedup vs baseline: 1.4912x; 1.0124x over previous
import functools

import jax
import jax.numpy as jnp
from jax import lax
from jax.experimental import pallas as pl
from jax.experimental.pallas import tpu as pltpu
from jax.experimental.pallas import tpu_sc as plsc

_NC = 2
_NS = 16
_NW = _NC * _NS


def _build(b, s, d, dtype):
    bw = b // _NW
    nbuf = 2

    mesh = plsc.VectorSubcoreMesh(core_axis_name="c", subcore_axis_name="s")

    @functools.partial(
        pl.kernel,
        out_type=jax.ShapeDtypeStruct((s, b, 2 * d), dtype),
        mesh=mesh,
        scratch_types=[
            pltpu.VMEM((s, bw), jnp.int32),
            [pltpu.VMEM((bw, 2 * d), dtype)] * nbuf,
            [pltpu.SemaphoreType.DMA] * nbuf,
        ],
        compiler_params=pltpu.CompilerParams(needs_layout_passes=False),
    )
    def emb(idx_hbm, table2_hbm, out_hbm, xbt, rows, gsems):
        wid = lax.axis_index("s") * _NC + lax.axis_index("c")
        pltpu.sync_copy(idx_hbm.at[:, pl.ds(wid * bw, bw)], xbt)

        for u in range(nbuf):
            pltpu.async_copy(table2_hbm.at[xbt.at[u]], rows[u], gsems[u])

        @pl.loop(0, s, step=nbuf)
        def _group(g):
            for u in range(nbuf):
                j = g + u
                pltpu.make_async_copy(
                    table2_hbm.at[xbt.at[j]], rows[u], gsems[u]
                ).wait()
                pltpu.sync_copy(rows[u], out_hbm.at[j, pl.ds(wid * bw, bw)])

                @pl.when(j + nbuf < s)
                def _prefetch():
                    pltpu.async_copy(
                        table2_hbm.at[xbt.at[j + nbuf]], rows[u], gsems[u]
                    )

    return emb


def kernel(x, table):
    b, s = x.shape
    v, d = table.shape
    xt = jnp.swapaxes(x, 0, 1).astype(jnp.int32)
    table2 = jnp.pad(table, ((0, 0), (0, d)))
    out4 = _build(b, s, d, table.dtype)(xt, table2)
    return out4[:, :, :d].transpose(1, 0, 2)


# speedup vs baseline: 1.4930x; 1.0012x over previous
import functools

import jax
import jax.numpy as jnp
from jax import lax
from jax.experimental import pallas as pl
from jax.experimental.pallas import tpu as pltpu
from jax.experimental.pallas import tpu_sc as plsc

_NC = 2
_NS = 16
_NW = _NC * _NS


def _build(b, s, d, dtype):
    b_per_w = b // _NW
    nbuf = 4

    mesh = plsc.VectorSubcoreMesh(core_axis_name="c", subcore_axis_name="s")

    @functools.partial(
        pl.kernel,
        out_type=jax.ShapeDtypeStruct((b, s, d), dtype),
        mesh=mesh,
        scratch_types=[
            pltpu.VMEM((b_per_w, s), jnp.int32),
            pltpu.VMEM((nbuf, s, d), dtype),
            [pltpu.SemaphoreType.DMA] * nbuf,
        ],
        compiler_params=pltpu.CompilerParams(use_tc_tiling_on_sc=False),
    )
    def emb(idx_hbm, table_hbm, out_hbm, idx_v, rows_v, gsems):
        wid = lax.axis_index("s") * _NC + lax.axis_index("c")
        base = wid * b_per_w
        pltpu.sync_copy(idx_hbm.at[pl.ds(base, b_per_w)], idx_v)

        for u in range(nbuf):
            pltpu.async_copy(table_hbm.at[idx_v.at[u]], rows_v.at[u], gsems[u])

        @pl.loop(0, b_per_w, step=nbuf)
        def _group(g):
            for u in range(nbuf):
                j = g + u
                pltpu.make_async_copy(
                    table_hbm.at[idx_v.at[j]], rows_v.at[u], gsems[u]
                ).wait()
                pltpu.sync_copy(rows_v.at[u], out_hbm.at[base + j])

                @pl.when(j + nbuf < b_per_w)
                def _prefetch():
                    pltpu.async_copy(
                        table_hbm.at[idx_v.at[j + nbuf]], rows_v.at[u], gsems[u]
                    )

    return emb


def kernel(x, table):
    b, s = x.shape
    v, d = table.shape
    return _build(b, s, d, table.dtype)(x.astype(jnp.int32), table)
